# baseline (speedup 1.0000x reference)
import functools

import jax
import jax.numpy as jnp
from jax import lax
from jax.experimental import pallas as pl
from jax.experimental.pallas import tpu as pltpu
from jax.experimental.pallas import tpu_sc as plsc

N = 10000
E = 320000
D = 128
C = 64

NC = 2
NS = 16
NW = NC * NS
EW = E // NW
CH = 128
EWP = 10240
NCH = EWP // CH
EP = NW * EWP
DH = D // 2
DH = D // 2
RPT0 = 624
RPTL = N - 15 * RPT0


@functools.cache
def _build_sc():
    mesh = plsc.VectorSubcoreMesh(
        core_axis_name="c", subcore_axis_name="s",
        num_cores=NC, num_subcores=NS)

    def _worker_id():
        return lax.axis_index("s") * NC + lax.axis_index("c")

    @functools.partial(
        pl.kernel,
        out_type=jax.ShapeDtypeStruct((NW, NCH, CH, D), jnp.float32),
        mesh=mesh,
        scratch_types=[
            pltpu.VMEM((NCH, CH), jnp.int32),
            pltpu.VMEM((CH, D), jnp.float32),
            pltpu.VMEM((CH, D), jnp.float32),
            pltpu.SemaphoreType.DMA,
            pltpu.SemaphoreType.DMA,
        ],
    )
    def _gather(u_hbm, src_hbm, out_hbm, src_v, rows0, rows1, sem0, sem1):
        wid = _worker_id()
        pltpu.sync_copy(src_hbm.at[wid], src_v)

        def start(i, rows, sem):
            pltpu.async_copy(u_hbm.at[src_v.at[i]], rows, sem)

        def wait(i, rows, sem):
            pltpu.make_async_copy(u_hbm.at[src_v.at[i]], rows, sem).wait()

        start(0, rows0, sem0)

        def body(it, carry):
            a = 2 * it
            wait(a, rows0, sem0)
            start(a + 1, rows1, sem1)
            pltpu.sync_copy(rows0, out_hbm.at[wid, a])
            start(a + 2, rows0, sem0)
            wait(a + 1, rows1, sem1)
            pltpu.sync_copy(rows1, out_hbm.at[wid, a + 1])
            return carry

        lax.fori_loop(0, (NCH - 2) // 2, body, 0)

        last = NCH - 2
        wait(last, rows0, sem0)
        start(last + 1, rows1, sem1)
        pltpu.sync_copy(rows0, out_hbm.at[wid, last])
        wait(last + 1, rows1, sem1)
        pltpu.sync_copy(rows1, out_hbm.at[wid, last + 1])

    @functools.partial(
        pl.kernel,
        out_type=jax.ShapeDtypeStruct((NC, N, D), jnp.float32),
        mesh=mesh,
        scratch_types=[
            pltpu.VMEM((NCH, CH), jnp.int32),
            pltpu.VMEM((CH, D), jnp.float32),
            pltpu.VMEM((CH, D), jnp.float32),
            pltpu.VMEM_SHARED((N, D), jnp.float32),
            pltpu.SemaphoreType.DMA,
            pltpu.SemaphoreType.DMA,
            pltpu.SemaphoreType.DMA,
            pltpu.SemaphoreType.DMA,
        ],
    )
    def _scatter(s_hbm, dst_hbm, zrows_hbm, out_hbm,
                 dst_v, half0, half1, shared, sem0, sem1, sems0, sems1):
        c = lax.axis_index("c")
        s = lax.axis_index("s")
        wid = _worker_id()
        pltpu.sync_copy(dst_hbm.at[wid], dst_v)

        @pl.when(s < 15)
        def _z0():
            pltpu.sync_copy(zrows_hbm.at[pl.ds(0, RPT0)],
                            shared.at[pl.ds(s * RPT0, RPT0)])

        @pl.when(s == 15)
        def _z1():
            pltpu.sync_copy(zrows_hbm, shared.at[pl.ds(15 * RPT0, RPTL)])

        plsc.subcore_barrier()

        def start(i, half, sem):
            pltpu.async_copy(s_hbm.at[wid, i], half, sem)

        def wait(i, half, sem):
            pltpu.make_async_copy(s_hbm.at[wid, i], half, sem).wait()

        def start_s(i, half, sem):
            pltpu.async_copy(half, shared.at[dst_v.at[i]], sem, add=True)

        def wait_s(i, half, sem):
            pltpu.make_async_copy(half, shared.at[dst_v.at[i]],
                                  sem).wait()

        start(0, half0, sem0)
        start(1, half1, sem1)

        def body(it, carry):
            a = 2 * it
            b = a + 1
            wait(a, half0, sem0)
            start_s(a, half0, sems0)
            wait(b, half1, sem1)
            start_s(b, half1, sems1)
            wait_s(a, half0, sems0)
            start(a + 2, half0, sem0)
            wait_s(b, half1, sems1)
            start(b + 2, half1, sem1)
            return carry

        lax.fori_loop(0, (NCH - 2) // 2, body, 0)

        last = NCH - 2
        wait(last, half0, sem0)
        start_s(last, half0, sems0)
        wait(last + 1, half1, sem1)
        start_s(last + 1, half1, sems1)
        wait_s(last, half0, sems0)
        wait_s(last + 1, half1, sems1)

        plsc.subcore_barrier()

        @pl.when(s < 15)
        def _o0():
            pltpu.sync_copy(shared.at[pl.ds(s * RPT0, RPT0)],
                            out_hbm.at[c, pl.ds(s * RPT0, RPT0)])

        @pl.when(s == 15)
        def _o1():
            pltpu.sync_copy(shared.at[pl.ds(15 * RPT0, RPTL)],
                            out_hbm.at[c, pl.ds(15 * RPT0, RPTL)])

    return _gather, _scatter


BN = 2000
BE = EWP


def _scale_tc(g, ew2):
    def k(g_ref, w_ref, o_ref):
        o_ref[...] = g_ref[...] * w_ref[...]

    return pl.pallas_call(
        k,
        grid=(EP // BE,),
        in_specs=[pl.BlockSpec((BE, D), lambda i: (i, 0)),
                  pl.BlockSpec((BE, 1), lambda i: (i, 0))],
        out_specs=pl.BlockSpec((BE, D), lambda i: (i, 0)),
        out_shape=jax.ShapeDtypeStruct((EP, D), jnp.float32),
    )(g, ew2)


def _wdeg_tc(ew2):
    def k(w_ref, o_ref):
        o_ref[...] = jnp.broadcast_to(w_ref[...], (BE, D))

    return pl.pallas_call(
        k,
        grid=(EP // BE,),
        in_specs=[pl.BlockSpec((BE, 1), lambda i: (i, 0))],
        out_specs=pl.BlockSpec((BE, D), lambda i: (i, 0)),
        out_shape=jax.ShapeDtypeStruct((EP, D), jnp.float32),
    )(ew2)


def _dis_tc(degp, x):
    def k(deg_ref, x_ref, dis_ref, u_ref):
        deg = deg_ref[0, :, 0] + deg_ref[1, :, 0]
        dis = jnp.where(deg > 0, lax.rsqrt(deg), 0.0)
        dis_ref[...] = dis[:, None]
        u_ref[...] = x_ref[...] * dis[:, None]

    return pl.pallas_call(
        k,
        grid=(1,),
        in_specs=[
            pl.BlockSpec((NC, N, D), lambda i: (0, 0, 0)),
            pl.BlockSpec((N, D), lambda i: (0, 0)),
        ],
        out_specs=[
            pl.BlockSpec((N, 1), lambda i: (0, 0)),
            pl.BlockSpec((N, D), lambda i: (0, 0)),
        ],
        out_shape=[jax.ShapeDtypeStruct((N, 1), jnp.float32),
                   jax.ShapeDtypeStruct((N, D), jnp.float32)],
    )(degp, x)


def _stage_a(A, dis2, h, W):
    def k(a_ref, d_ref, h_ref, w_ref, u2_ref, s_ref):
        dis = d_ref[...]
        t1 = -(a_ref[0] + a_ref[1]) * dis
        u2_ref[...] = t1 * dis
        s_ref[...] = (
            jnp.dot(h_ref[...], w_ref[0], preferred_element_type=jnp.float32)
            + jnp.dot(t1, w_ref[1], preferred_element_type=jnp.float32))

    return pl.pallas_call(
        k,
        grid=(N // BN,),
        in_specs=[
            pl.BlockSpec((NC, BN, D), lambda i: (0, i, 0)),
            pl.BlockSpec((BN, 1), lambda i: (i, 0)),
            pl.BlockSpec((BN, D), lambda i: (i, 0)),
            pl.BlockSpec((3, D, D), lambda i: (0, 0, 0)),
        ],
        out_specs=[
            pl.BlockSpec((BN, D), lambda i: (i, 0)),
            pl.BlockSpec((BN, D), lambda i: (i, 0)),
        ],
        out_shape=[
            jax.ShapeDtypeStruct((N, D), jnp.float32),
            jax.ShapeDtypeStruct((N, D), jnp.float32),
        ],
    )(A, dis2, h, W)


def _stage_b(B, dis2, h, S, W, bias):
    def k(b_ref, d_ref, h_ref, s_ref, w_ref, bias_ref, out_ref, uh_ref):
        dis = d_ref[...]
        t2 = -2.0 * (b_ref[0] + b_ref[1]) * dis - h_ref[...]
        hh = (s_ref[...]
              + jnp.dot(t2, w_ref[2], preferred_element_type=jnp.float32)
              + bias_ref[...])
        hh = jnp.maximum(hh, 0.0)
        out_ref[...] = hh
        uh_ref[...] = hh * dis

    return pl.pallas_call(
        k,
        grid=(N // BN,),
        in_specs=[
            pl.BlockSpec((NC, BN, D), lambda i: (0, i, 0)),
            pl.BlockSpec((BN, 1), lambda i: (i, 0)),
            pl.BlockSpec((BN, D), lambda i: (i, 0)),
            pl.BlockSpec((BN, D), lambda i: (i, 0)),
            pl.BlockSpec((3, D, D), lambda i: (0, 0, 0)),
            pl.BlockSpec((1, D), lambda i: (0, 0)),
        ],
        out_specs=[
            pl.BlockSpec((BN, D), lambda i: (i, 0)),
            pl.BlockSpec((BN, D), lambda i: (i, 0)),
        ],
        out_shape=[
            jax.ShapeDtypeStruct((N, D), jnp.float32),
            jax.ShapeDtypeStruct((N, D), jnp.float32),
        ],
    )(B, dis2, h, S, W, bias)


def _head(H, Wout, bout):
    def k(h_ref, w_ref, b_ref, o_ref):
        logits = (jnp.dot(h_ref[...], w_ref[...],
                          preferred_element_type=jnp.float32) + b_ref[...])
        m = jnp.max(logits, axis=1, keepdims=True)
        z = logits - m
        lse = jnp.log(jnp.sum(jnp.exp(z), axis=1, keepdims=True))
        o_ref[...] = z - lse

    return pl.pallas_call(
        k,
        grid=(N // BN,),
        in_specs=[
            pl.BlockSpec((BN, D), lambda i: (i, 0)),
            pl.BlockSpec((D, C), lambda i: (0, 0)),
            pl.BlockSpec((1, C), lambda i: (0, 0)),
        ],
        out_specs=pl.BlockSpec((BN, C), lambda i: (i, 0)),
        out_shape=jax.ShapeDtypeStruct((N, C), jnp.float32),
    )(H, Wout, bout)


def kernel(x, edge_index, edge_weights, W1, b1, W2, b2, Wout, bout):
    pad_i = jnp.zeros((NW, EWP - EW), jnp.int32)
    src = jnp.concatenate(
        [edge_index[0].reshape(NW, EW), pad_i], axis=1).reshape(NW, NCH, CH)
    dst = jnp.concatenate(
        [edge_index[1].reshape(NW, EW), pad_i], axis=1).reshape(NW, NCH, CH)
    ew_pad = jnp.concatenate(
        [edge_weights.reshape(NW, EW),
         jnp.zeros((NW, EWP - EW), jnp.float32)], axis=1).reshape(EP, 1)
    zrows = jnp.zeros((RPTL, D), jnp.float32)

    _gather, _scatter = _build_sc()

    def prop(u):
        g = _gather(u, src).reshape(EP, D)
        sc = _scale_tc(g, ew_pad)
        return _scatter(sc.reshape(NW, NCH, CH, D), dst, zrows)

    wb = _wdeg_tc(ew_pad).reshape(NW, NCH, CH, D)
    degp = _scatter(wb, src, zrows)
    dis2, u = _dis_tc(degp, x)

    h = x
    for W, b in ((W1, b1), (W2, b2)):
        A = prop(u)
        u, S = _stage_a(A, dis2, h, W)
        B = prop(u)
        h, u = _stage_b(B, dis2, h, S, W, b.reshape(1, D))

    return _head(h, Wout, bout.reshape(1, C))


# speedup vs baseline: 1.0570x; 1.0570x over previous
import functools

import jax
import jax.numpy as jnp
from jax import lax
from jax.experimental import pallas as pl
from jax.experimental.pallas import tpu as pltpu
from jax.experimental.pallas import tpu_sc as plsc

N = 10000
E = 320000
D = 128
C = 64

NC = 2
NS = 16
NW = NC * NS
EW = E // NW
CH = 128
EWP = 10240
NCH = EWP // CH
EP = NW * EWP
DH = D // 2
DH = D // 2
RPT0 = 624
RPTL = N - 15 * RPT0


@functools.cache
def _build_sc():
    mesh = plsc.VectorSubcoreMesh(
        core_axis_name="c", subcore_axis_name="s",
        num_cores=NC, num_subcores=NS)

    def _worker_id():
        return lax.axis_index("s") * NC + lax.axis_index("c")

    @functools.partial(
        pl.kernel,
        out_type=jax.ShapeDtypeStruct((NW, NCH, CH, D), jnp.float32),
        mesh=mesh,
        scratch_types=[
            pltpu.VMEM((NCH, CH), jnp.int32),
            pltpu.VMEM((CH, D), jnp.float32),
            pltpu.VMEM((CH, D), jnp.float32),
            pltpu.SemaphoreType.DMA,
            pltpu.SemaphoreType.DMA,
        ],
    )
    def _gather(u_hbm, src_hbm, out_hbm, src_v, rows0, rows1, sem0, sem1):
        wid = _worker_id()
        pltpu.sync_copy(src_hbm.at[wid], src_v)

        def start(i, rows, sem):
            pltpu.async_copy(u_hbm.at[src_v.at[i]], rows, sem)

        def wait(i, rows, sem):
            pltpu.make_async_copy(u_hbm.at[src_v.at[i]], rows, sem).wait()

        start(0, rows0, sem0)

        def body(it, carry):
            a = 2 * it
            wait(a, rows0, sem0)
            start(a + 1, rows1, sem1)
            pltpu.sync_copy(rows0, out_hbm.at[wid, a])
            start(a + 2, rows0, sem0)
            wait(a + 1, rows1, sem1)
            pltpu.sync_copy(rows1, out_hbm.at[wid, a + 1])
            return carry

        lax.fori_loop(0, (NCH - 2) // 2, body, 0)

        last = NCH - 2
        wait(last, rows0, sem0)
        start(last + 1, rows1, sem1)
        pltpu.sync_copy(rows0, out_hbm.at[wid, last])
        wait(last + 1, rows1, sem1)
        pltpu.sync_copy(rows1, out_hbm.at[wid, last + 1])

    @functools.partial(
        pl.kernel,
        out_type=jax.ShapeDtypeStruct((NC, N, D), jnp.float32),
        mesh=mesh,
        scratch_types=[
            pltpu.VMEM((NCH, CH), jnp.int32),
            pltpu.VMEM((CH, D), jnp.float32),
            pltpu.VMEM((CH, D), jnp.float32),
            pltpu.VMEM_SHARED((N, D), jnp.float32),
            pltpu.SemaphoreType.DMA,
            pltpu.SemaphoreType.DMA,
        ],
    )
    def _scatter(s_hbm, dst_hbm, zrows_hbm, out_hbm,
                 dst_v, half0, half1, shared, sem0, sem1):
        c = lax.axis_index("c")
        s = lax.axis_index("s")
        wid = _worker_id()
        pltpu.sync_copy(dst_hbm.at[wid], dst_v)

        @pl.when(s < 15)
        def _z0():
            pltpu.sync_copy(zrows_hbm.at[pl.ds(0, RPT0)],
                            shared.at[pl.ds(s * RPT0, RPT0)])

        @pl.when(s == 15)
        def _z1():
            pltpu.sync_copy(zrows_hbm, shared.at[pl.ds(15 * RPT0, RPTL)])

        plsc.subcore_barrier()

        def start(i, half, sem):
            pltpu.async_copy(s_hbm.at[wid, i], half, sem)

        def wait(i, half, sem):
            pltpu.make_async_copy(s_hbm.at[wid, i], half, sem).wait()

        start(0, half0, sem0)

        def body(it, carry):
            a = 2 * it
            wait(a, half0, sem0)
            start(a + 1, half1, sem1)
            pltpu.sync_copy(half0, shared.at[dst_v.at[a]], add=True)
            start(a + 2, half0, sem0)
            wait(a + 1, half1, sem1)
            pltpu.sync_copy(half1, shared.at[dst_v.at[a + 1]], add=True)
            return carry

        lax.fori_loop(0, (NCH - 2) // 2, body, 0)

        last = NCH - 2
        wait(last, half0, sem0)
        start(last + 1, half1, sem1)
        pltpu.sync_copy(half0, shared.at[dst_v.at[last]], add=True)
        wait(last + 1, half1, sem1)
        pltpu.sync_copy(half1, shared.at[dst_v.at[last + 1]], add=True)

        plsc.subcore_barrier()

        @pl.when(s < 15)
        def _o0():
            pltpu.sync_copy(shared.at[pl.ds(s * RPT0, RPT0)],
                            out_hbm.at[c, pl.ds(s * RPT0, RPT0)])

        @pl.when(s == 15)
        def _o1():
            pltpu.sync_copy(shared.at[pl.ds(15 * RPT0, RPTL)],
                            out_hbm.at[c, pl.ds(15 * RPT0, RPTL)])

    return _gather, _scatter


BN = 2000
BE = EWP


def _scale_tc(g, ew2):
    def k(g_ref, w_ref, o_ref):
        o_ref[...] = g_ref[...] * w_ref[...]

    return pl.pallas_call(
        k,
        grid=(EP // BE,),
        in_specs=[pl.BlockSpec((BE, D), lambda i: (i, 0)),
                  pl.BlockSpec((BE, 1), lambda i: (i, 0))],
        out_specs=pl.BlockSpec((BE, D), lambda i: (i, 0)),
        out_shape=jax.ShapeDtypeStruct((EP, D), jnp.float32),
    )(g, ew2)


def _wdeg_tc(ew2):
    def k(w_ref, o_ref):
        o_ref[...] = jnp.broadcast_to(w_ref[...], (BE, D))

    return pl.pallas_call(
        k,
        grid=(EP // BE,),
        in_specs=[pl.BlockSpec((BE, 1), lambda i: (i, 0))],
        out_specs=pl.BlockSpec((BE, D), lambda i: (i, 0)),
        out_shape=jax.ShapeDtypeStruct((EP, D), jnp.float32),
    )(ew2)


def _dis_tc(degp, x):
    def k(deg_ref, x_ref, dis_ref, u_ref):
        deg = deg_ref[0, :, 0] + deg_ref[1, :, 0]
        dis = jnp.where(deg > 0, lax.rsqrt(deg), 0.0)
        dis_ref[...] = dis[:, None]
        u_ref[...] = x_ref[...] * dis[:, None]

    return pl.pallas_call(
        k,
        grid=(1,),
        in_specs=[
            pl.BlockSpec((NC, N, D), lambda i: (0, 0, 0)),
            pl.BlockSpec((N, D), lambda i: (0, 0)),
        ],
        out_specs=[
            pl.BlockSpec((N, 1), lambda i: (0, 0)),
            pl.BlockSpec((N, D), lambda i: (0, 0)),
        ],
        out_shape=[jax.ShapeDtypeStruct((N, 1), jnp.float32),
                   jax.ShapeDtypeStruct((N, D), jnp.float32)],
    )(degp, x)


def _stage_a(A, dis2, h, W):
    def k(a_ref, d_ref, h_ref, w_ref, u2_ref, s_ref):
        dis = d_ref[...]
        t1 = -(a_ref[0] + a_ref[1]) * dis
        u2_ref[...] = t1 * dis
        s_ref[...] = (
            jnp.dot(h_ref[...], w_ref[0], preferred_element_type=jnp.float32)
            + jnp.dot(t1, w_ref[1], preferred_element_type=jnp.float32))

    return pl.pallas_call(
        k,
        grid=(N // BN,),
        in_specs=[
            pl.BlockSpec((NC, BN, D), lambda i: (0, i, 0)),
            pl.BlockSpec((BN, 1), lambda i: (i, 0)),
            pl.BlockSpec((BN, D), lambda i: (i, 0)),
            pl.BlockSpec((3, D, D), lambda i: (0, 0, 0)),
        ],
        out_specs=[
            pl.BlockSpec((BN, D), lambda i: (i, 0)),
            pl.BlockSpec((BN, D), lambda i: (i, 0)),
        ],
        out_shape=[
            jax.ShapeDtypeStruct((N, D), jnp.float32),
            jax.ShapeDtypeStruct((N, D), jnp.float32),
        ],
    )(A, dis2, h, W)


def _stage_b(B, dis2, h, S, W, bias):
    def k(b_ref, d_ref, h_ref, s_ref, w_ref, bias_ref, out_ref, uh_ref):
        dis = d_ref[...]
        t2 = -2.0 * (b_ref[0] + b_ref[1]) * dis - h_ref[...]
        hh = (s_ref[...]
              + jnp.dot(t2, w_ref[2], preferred_element_type=jnp.float32)
              + bias_ref[...])
        hh = jnp.maximum(hh, 0.0)
        out_ref[...] = hh
        uh_ref[...] = hh * dis

    return pl.pallas_call(
        k,
        grid=(N // BN,),
        in_specs=[
            pl.BlockSpec((NC, BN, D), lambda i: (0, i, 0)),
            pl.BlockSpec((BN, 1), lambda i: (i, 0)),
            pl.BlockSpec((BN, D), lambda i: (i, 0)),
            pl.BlockSpec((BN, D), lambda i: (i, 0)),
            pl.BlockSpec((3, D, D), lambda i: (0, 0, 0)),
            pl.BlockSpec((1, D), lambda i: (0, 0)),
        ],
        out_specs=[
            pl.BlockSpec((BN, D), lambda i: (i, 0)),
            pl.BlockSpec((BN, D), lambda i: (i, 0)),
        ],
        out_shape=[
            jax.ShapeDtypeStruct((N, D), jnp.float32),
            jax.ShapeDtypeStruct((N, D), jnp.float32),
        ],
    )(B, dis2, h, S, W, bias)


def _head(H, Wout, bout):
    def k(h_ref, w_ref, b_ref, o_ref):
        logits = (jnp.dot(h_ref[...], w_ref[...],
                          preferred_element_type=jnp.float32) + b_ref[...])
        m = jnp.max(logits, axis=1, keepdims=True)
        z = logits - m
        lse = jnp.log(jnp.sum(jnp.exp(z), axis=1, keepdims=True))
        o_ref[...] = z - lse

    return pl.pallas_call(
        k,
        grid=(N // BN,),
        in_specs=[
            pl.BlockSpec((BN, D), lambda i: (i, 0)),
            pl.BlockSpec((D, C), lambda i: (0, 0)),
            pl.BlockSpec((1, C), lambda i: (0, 0)),
        ],
        out_specs=pl.BlockSpec((BN, C), lambda i: (i, 0)),
        out_shape=jax.ShapeDtypeStruct((N, C), jnp.float32),
    )(H, Wout, bout)


def kernel(x, edge_index, edge_weights, W1, b1, W2, b2, Wout, bout):
    pad_i = jnp.zeros((NW, EWP - EW), jnp.int32)
    src = jnp.concatenate(
        [edge_index[0].reshape(NW, EW), pad_i], axis=1).reshape(NW, NCH, CH)
    dst = jnp.concatenate(
        [edge_index[1].reshape(NW, EW), pad_i], axis=1).reshape(NW, NCH, CH)
    ew_pad = jnp.concatenate(
        [edge_weights.reshape(NW, EW),
         jnp.zeros((NW, EWP - EW), jnp.float32)], axis=1).reshape(EP, 1)
    zrows = jnp.zeros((RPTL, D), jnp.float32)

    _gather, _scatter = _build_sc()

    def prop(u):
        g = _gather(u, src).reshape(EP, D)
        sc = _scale_tc(g, ew_pad)
        return _scatter(sc.reshape(NW, NCH, CH, D), dst, zrows)

    wb = _wdeg_tc(ew_pad).reshape(NW, NCH, CH, D)
    degp = _scatter(wb, src, zrows)
    dis2, u = _dis_tc(degp, x)

    h = x
    for W, b in ((W1, b1), (W2, b2)):
        A = prop(u)
        u, S = _stage_a(A, dis2, h, W)
        B = prop(u)
        h, u = _stage_b(B, dis2, h, S, W, b.reshape(1, D))

    return _head(h, Wout, bout.reshape(1, C))
